# Initial kernel scaffold; baseline (speedup 1.0000x reference)
import jax
import jax.numpy as jnp
from jax.experimental import pallas as pl


def kernel(x, weights, channel_idx):
    raise NotImplementedError("write your pallas kernel here")


# speedup vs baseline: 1.0708x; 1.0708x over previous
import jax
import jax.numpy as jnp
from jax.experimental import pallas as pl

_B, _C, _H, _W = 8, 128, 128, 128
_F = 32
_HW = _H * _W
_S = 4096


def _gemm_kernel(idx_ref, w_ref, x_ref, o_ref):
    idxv = idx_ref[0, :]
    cols = jax.lax.broadcasted_iota(jnp.int32, (_C, _C), 1)
    onehot_t = (idxv[:, None] == cols).astype(jnp.float32)
    w_eff = jnp.dot(w_ref[...], onehot_t, preferred_element_type=jnp.float32)
    o_ref[0] = jnp.dot(w_eff, x_ref[0], preferred_element_type=jnp.float32)


def kernel(x, weights, channel_idx):
    xf = x.reshape(_B, _C, _HW)
    w2 = weights.reshape(_F, _C)
    idx2 = channel_idx.reshape(1, _C)
    out = pl.pallas_call(
        _gemm_kernel,
        grid=(_B, _HW // _S),
        in_specs=[
            pl.BlockSpec((1, _C), lambda b, s: (0, 0)),
            pl.BlockSpec((_F, _C), lambda b, s: (0, 0)),
            pl.BlockSpec((1, _C, _S), lambda b, s: (b, 0, s)),
        ],
        out_specs=pl.BlockSpec((1, _F, _S), lambda b, s: (b, 0, s)),
        out_shape=jax.ShapeDtypeStruct((_B, _F, _HW), jnp.float32),
    )(idx2, w2, xf)
    return out.reshape(_B, _F, _H, _W)
